# baseline (device time: 18666 ns/iter reference)
import jax
import jax.numpy as jnp
from jax import lax
from jax.experimental import pallas as pl
from jax.experimental.pallas import tpu as pltpu

N_DEV = 8
N_TOK = 512
D_IN = 256
D_OUT = 512
N_EXPERTS = 16
ROWS = N_TOK // N_DEV


def kernel(x, router_W, route_idx, expert_W):
    def body(x_ref, rw_ref, idx_ref, ew_ref, out_ref,
             partial_ref, gather_ref, send_sems, recv_sems):
        my = lax.axis_index("i")

        xx = x_ref[...]
        scores = jnp.dot(xx, rw_ref[...], preferred_element_type=jnp.float32)
        m = jnp.max(scores, axis=-1, keepdims=True)
        p = jnp.exp(scores - m)
        idx0 = idx_ref[:, 0:1]
        idx1 = idx_ref[:, 1:2]
        ecols = lax.broadcasted_iota(jnp.int32, (N_TOK, N_EXPERTS), 1)
        g0 = jnp.sum(jnp.where(ecols == idx0, p, 0.0), axis=1, keepdims=True)
        g1 = jnp.sum(jnp.where(ecols == idx1, p, 0.0), axis=1, keepdims=True)
        gs = g0 + g1
        e0 = 2 * my
        e1 = 2 * my + 1
        w0 = (jnp.where(idx0 == e0, g0, 0.0) + jnp.where(idx1 == e0, g1, 0.0)) / gs
        w1 = (jnp.where(idx0 == e1, g0, 0.0) + jnp.where(idx1 == e1, g1, 0.0)) / gs
        partial_ref[...] = (
            jnp.dot(xx * w0, ew_ref[0], preferred_element_type=jnp.float32)
            + jnp.dot(xx * w1, ew_ref[1], preferred_element_type=jnp.float32)
        )

        barrier = pltpu.get_barrier_semaphore()
        for o in range(1, N_DEV):
            peer = lax.rem(my + o, N_DEV)
            pl.semaphore_signal(barrier, inc=1, device_id=(peer,),
                                device_id_type=pl.DeviceIdType.MESH)
        pl.semaphore_wait(barrier, N_DEV - 1)

        rdmas = []
        for o in range(1, N_DEV):
            tgt = lax.rem(my + o, N_DEV)
            rdma = pltpu.make_async_remote_copy(
                src_ref=partial_ref.at[pl.ds(tgt * ROWS, ROWS), :],
                dst_ref=gather_ref.at[o],
                send_sem=send_sems.at[o],
                recv_sem=recv_sems.at[o],
                device_id=(tgt,),
                device_id_type=pl.DeviceIdType.MESH,
            )
            rdma.start()
            rdmas.append(rdma)

        gather_ref[0] = partial_ref[pl.ds(my * ROWS, ROWS), :]

        for rdma in rdmas:
            rdma.wait()

        out_ref[...] = jnp.sum(gather_ref[...], axis=0)

    return pl.pallas_call(
        body,
        out_shape=jax.ShapeDtypeStruct((ROWS, D_OUT), jnp.float32),
        in_specs=[
            pl.BlockSpec(memory_space=pltpu.VMEM),
            pl.BlockSpec(memory_space=pltpu.VMEM),
            pl.BlockSpec(memory_space=pltpu.VMEM),
            pl.BlockSpec(memory_space=pltpu.VMEM),
        ],
        out_specs=pl.BlockSpec(memory_space=pltpu.VMEM),
        scratch_shapes=[
            pltpu.VMEM((N_TOK, D_OUT), jnp.float32),
            pltpu.VMEM((N_DEV, ROWS, D_OUT), jnp.float32),
            pltpu.SemaphoreType.DMA((N_DEV,)),
            pltpu.SemaphoreType.DMA((N_DEV,)),
        ],
        compiler_params=pltpu.CompilerParams(collective_id=0),
    )(x, router_W, route_idx, expert_W)


# device time: 16662 ns/iter; 1.1203x vs baseline; 1.1203x over previous
import jax
import jax.numpy as jnp
from jax import lax
from jax.experimental import pallas as pl
from jax.experimental.pallas import tpu as pltpu

N_DEV = 8
N_TOK = 512
D_IN = 256
D_OUT = 512
N_EXPERTS = 16
ROWS = N_TOK // N_DEV


def kernel(x, router_W, route_idx, expert_W):
    def body(x_ref, rw_ref, idx_ref, ew_ref, out_ref,
             xw_ref, send_ref, gather_ref, send_sems, recv_sems):
        my = lax.axis_index("i")

        barrier = pltpu.get_barrier_semaphore()
        for o in range(1, N_DEV):
            peer = lax.rem(my + o, N_DEV)
            pl.semaphore_signal(barrier, inc=1, device_id=(peer,),
                                device_id_type=pl.DeviceIdType.MESH)

        xx = x_ref[...]
        scores = jnp.dot(xx, rw_ref[...], preferred_element_type=jnp.float32)
        m = jnp.max(scores, axis=-1, keepdims=True)
        p = jnp.exp(scores - m)
        idx0 = idx_ref[:, 0:1]
        idx1 = idx_ref[:, 1:2]
        ecols = lax.broadcasted_iota(jnp.int32, (N_TOK, N_EXPERTS), 1)
        g0 = jnp.sum(jnp.where(ecols == idx0, p, 0.0), axis=1, keepdims=True)
        g1 = jnp.sum(jnp.where(ecols == idx1, p, 0.0), axis=1, keepdims=True)
        gs = g0 + g1
        e0 = 2 * my
        e1 = 2 * my + 1
        w0 = (jnp.where(idx0 == e0, g0, 0.0) + jnp.where(idx1 == e0, g1, 0.0)) / gs
        w1 = (jnp.where(idx0 == e1, g0, 0.0) + jnp.where(idx1 == e1, g1, 0.0)) / gs
        xw_ref[0] = xx * w0
        xw_ref[1] = xx * w1

        pl.semaphore_wait(barrier, N_DEV - 1)

        rdmas = []
        for o in range(1, N_DEV):
            tgt = lax.rem(my + o, N_DEV)
            a0 = xw_ref[0, pl.ds(tgt * ROWS, ROWS), :]
            a1 = xw_ref[1, pl.ds(tgt * ROWS, ROWS), :]
            send_ref[o] = (
                jnp.dot(a0, ew_ref[0], preferred_element_type=jnp.float32)
                + jnp.dot(a1, ew_ref[1], preferred_element_type=jnp.float32)
            )
            rdma = pltpu.make_async_remote_copy(
                src_ref=send_ref.at[o],
                dst_ref=gather_ref.at[o],
                send_sem=send_sems.at[o],
                recv_sem=recv_sems.at[o],
                device_id=(tgt,),
                device_id_type=pl.DeviceIdType.MESH,
            )
            rdma.start()
            rdmas.append(rdma)

        b0 = xw_ref[0, pl.ds(my * ROWS, ROWS), :]
        b1 = xw_ref[1, pl.ds(my * ROWS, ROWS), :]
        gather_ref[0] = (
            jnp.dot(b0, ew_ref[0], preferred_element_type=jnp.float32)
            + jnp.dot(b1, ew_ref[1], preferred_element_type=jnp.float32)
        )

        for rdma in rdmas:
            rdma.wait()

        out_ref[...] = jnp.sum(gather_ref[...], axis=0)

    return pl.pallas_call(
        body,
        out_shape=jax.ShapeDtypeStruct((ROWS, D_OUT), jnp.float32),
        in_specs=[
            pl.BlockSpec(memory_space=pltpu.VMEM),
            pl.BlockSpec(memory_space=pltpu.VMEM),
            pl.BlockSpec(memory_space=pltpu.VMEM),
            pl.BlockSpec(memory_space=pltpu.VMEM),
        ],
        out_specs=pl.BlockSpec(memory_space=pltpu.VMEM),
        scratch_shapes=[
            pltpu.VMEM((2, N_TOK, D_IN), jnp.float32),
            pltpu.VMEM((N_DEV, ROWS, D_OUT), jnp.float32),
            pltpu.VMEM((N_DEV, ROWS, D_OUT), jnp.float32),
            pltpu.SemaphoreType.DMA((N_DEV,)),
            pltpu.SemaphoreType.DMA((N_DEV,)),
        ],
        compiler_params=pltpu.CompilerParams(collective_id=0),
    )(x, router_W, route_idx, expert_W)
